# Initial kernel scaffold; baseline (speedup 1.0000x reference)
import jax
import jax.numpy as jnp
from jax.experimental import pallas as pl


def kernel(inputs, token_table, pos_table):
    raise NotImplementedError("write your pallas kernel here")


# speedup vs baseline: 1.0377x; 1.0377x over previous
import functools

import jax
import jax.numpy as jnp
from jax import lax
from jax.experimental import pallas as pl
from jax.experimental.pallas import tpu as pltpu
from jax.experimental.pallas import tpu_sc as plsc

VOCAB = 100000
SEQ_LEN = 2048
EMBED = 128
BATCH = 4

NC = 2
NS = 16
NW = NC * NS
ROWS = BATCH * SEQ_LEN
B_PER_W = ROWS // NW
GCHUNK = 128
NG = B_PER_W // GCHUNK
LANES = 16
SCALE = 11.31370849898476


def _sc_embed(idx2d, token_table, pos_table):
  mesh = plsc.VectorSubcoreMesh(core_axis_name="c", subcore_axis_name="s")

  @functools.partial(
      pl.kernel,
      mesh=mesh,
      out_type=jax.ShapeDtypeStruct((ROWS, EMBED), jnp.float32),
      scratch_types=[
          pltpu.VMEM((NG, GCHUNK), jnp.int32),
          pltpu.VMEM((B_PER_W, EMBED), jnp.float32),
          pltpu.VMEM((B_PER_W, EMBED), jnp.float32),
          pltpu.SemaphoreType.DMA,
      ],
  )
  def k(idx_hbm, tok_hbm, pos_hbm, out_hbm, idx_v, tok_v, pos_v, sem):
    wid = lax.axis_index("s") * NC + lax.axis_index("c")
    base = wid * B_PER_W
    pltpu.sync_copy(idx_hbm.at[pl.ds(wid * NG, NG)], idx_v)
    copies = []
    for j in range(NG):
      copies.append(
          pltpu.async_copy(
              tok_hbm.at[idx_v.at[j]],
              tok_v.at[pl.ds(j * GCHUNK, GCHUNK)],
              sem,
          ))
    pos_base = base % SEQ_LEN
    pltpu.sync_copy(pos_hbm.at[pl.ds(pos_base, B_PER_W)], pos_v)
    for c in copies:
      c.wait()

    def row(r, carry):
      for j in range(EMBED // LANES):
        sl = (r, pl.ds(j * LANES, LANES))
        tok_v[sl] = tok_v[sl] * SCALE + pos_v[sl]
      return carry

    lax.fori_loop(0, B_PER_W, row, 0, unroll=2)

    pltpu.sync_copy(tok_v, out_hbm.at[pl.ds(base, B_PER_W)])

  return k(idx2d, token_table, pos_table)


def kernel(inputs, token_table, pos_table):
  idx2d = inputs.reshape(NW * NG, GCHUNK).astype(jnp.int32)
  out = _sc_embed(idx2d, token_table, pos_table)
  return out.reshape(BATCH, SEQ_LEN, EMBED)


# speedup vs baseline: 1.3210x; 1.2730x over previous
import functools

import jax
import jax.numpy as jnp
from jax import lax
from jax.experimental import pallas as pl
from jax.experimental.pallas import tpu as pltpu
from jax.experimental.pallas import tpu_sc as plsc

VOCAB = 100000
SEQ_LEN = 2048
EMBED = 128
BATCH = 4

NC = 2
NS = 16
NW = NC * NS
ROWS = BATCH * SEQ_LEN
B_PER_W = ROWS // NW
CH = 64
NCH = B_PER_W // CH
LANES = 16
SCALE = 11.31370849898476


def _sc_embed(idx2d, token_table, pos_table):
  mesh = plsc.VectorSubcoreMesh(core_axis_name="c", subcore_axis_name="s")

  @functools.partial(
      pl.kernel,
      mesh=mesh,
      out_type=jax.ShapeDtypeStruct((ROWS, EMBED), jnp.float32),
      scratch_types=[
          pltpu.VMEM((NCH, CH), jnp.int32),
          pltpu.VMEM((B_PER_W, EMBED), jnp.float32),
          pltpu.VMEM((B_PER_W, EMBED), jnp.float32),
          pltpu.SemaphoreType.DMA((NCH,)),
          pltpu.SemaphoreType.DMA((NCH,)),
      ],
  )
  def k(idx_hbm, tok_hbm, pos_hbm, out_hbm, idx_v, tok_v, pos_v, gsem, wsem):
    wid = lax.axis_index("s") * NC + lax.axis_index("c")
    base = wid * B_PER_W
    pltpu.sync_copy(idx_hbm.at[pl.ds(wid * NCH, NCH)], idx_v)
    gathers = []
    for c in range(NCH):
      gathers.append(
          pltpu.async_copy(
              tok_hbm.at[idx_v.at[c]],
              tok_v.at[pl.ds(c * CH, CH)],
              gsem.at[c],
          ))
    pos_base = base % SEQ_LEN
    pltpu.sync_copy(pos_hbm.at[pl.ds(pos_base, B_PER_W)], pos_v)

    writes = []
    for c in range(NCH):
      gathers[c].wait()

      def row(r, carry):
        for j in range(EMBED // LANES):
          sl = (r, pl.ds(j * LANES, LANES))
          plsc.addupdate(pos_v.at[sl], tok_v[sl] * SCALE)
        return carry

      lax.fori_loop(c * CH, (c + 1) * CH, row, 0, unroll=2)
      writes.append(
          pltpu.async_copy(
              pos_v.at[pl.ds(c * CH, CH)],
              out_hbm.at[pl.ds(base + c * CH, CH)],
              wsem.at[c],
          ))
    for w in writes:
      w.wait()

  return k(idx2d, token_table, pos_table)


def kernel(inputs, token_table, pos_table):
  idx2d = inputs.reshape(NW * NCH, CH).astype(jnp.int32)
  out = _sc_embed(idx2d, token_table, pos_table)
  return out.reshape(BATCH, SEQ_LEN, EMBED)
